# baseline (speedup 1.0000x reference)
import functools

import jax
import jax.numpy as jnp
from jax.experimental import pallas as pl
from jax.experimental.pallas import tpu as pltpu

N = 10000
NUM_SYMPS = 360
ROW_BLOCK = 400

def _dot(a, b):
    return jnp.dot(a.astype(jnp.bfloat16), b.astype(jnp.bfloat16),
                   preferred_element_type=jnp.float32)


def _s1_kernel(x_ref, w1_ref, s1_ref):
    s1_ref[...] = _dot(x_ref[...], w1_ref[...])


def _pass1_kernel(s1_ref, b1_ref, w2_ref, adj_ref, s2_ref):
    h = jnp.maximum(_dot(adj_ref[...], s1_ref[...]) + b1_ref[...], 0.0)
    s2_ref[...] = _dot(h, w2_ref[...])


def _pass2_kernel(s2_ref, b2_ref, wsh_t_ref, bsh_ref, whc_t_ref, bhc_ref,
                  adj_ref, out_ref):
    h2 = _dot(adj_ref[...], s2_ref[...]) + b2_ref[...]
    logits_s = _dot(h2, wsh_t_ref[...]) + bsh_ref[...]
    logits_h = _dot(h2, whc_t_ref[...]) + bhc_ref[...]
    rows = (pl.program_id(0) * ROW_BLOCK
            + jax.lax.broadcasted_iota(jnp.int32, (ROW_BLOCK, 1), 0))
    out_ref[...] = jax.nn.sigmoid(
        jnp.where(rows < NUM_SYMPS, logits_s, logits_h))


@jax.jit
def kernel(x, adj, W1, b1, W2, b2, Wsh, bsh, Whc, bhc):
    nfeat = x.shape[1]
    nhid = W1.shape[1]
    dim = W2.shape[1]
    nherbs = Wsh.shape[0]
    num_blocks = N // ROW_BLOCK

    s1 = pl.pallas_call(
        _s1_kernel,
        out_shape=jax.ShapeDtypeStruct((N, nhid), jnp.float32),
    )(x, W1)

    full = lambda shape: pl.BlockSpec(shape, lambda i: (0, 0))

    s2 = pl.pallas_call(
        _pass1_kernel,
        grid=(num_blocks,),
        in_specs=[
            full((N, nhid)),
            full((1, nhid)),
            full((nhid, dim)),
            pl.BlockSpec((ROW_BLOCK, N), lambda i: (i, 0)),
        ],
        out_specs=pl.BlockSpec((ROW_BLOCK, dim), lambda i: (i, 0)),
        out_shape=jax.ShapeDtypeStruct((N, dim), jnp.float32),
        compiler_params=pltpu.CompilerParams(
            dimension_semantics=("parallel",)),
    )(s1, b1.reshape(1, nhid), W2, adj)

    out = pl.pallas_call(
        _pass2_kernel,
        grid=(num_blocks,),
        in_specs=[
            full((N, dim)),
            full((1, dim)),
            full((dim, nherbs)),
            full((1, nherbs)),
            full((dim, nherbs)),
            full((1, nherbs)),
            pl.BlockSpec((ROW_BLOCK, N), lambda i: (i, 0)),
        ],
        out_specs=pl.BlockSpec((ROW_BLOCK, nherbs), lambda i: (i, 0)),
        out_shape=jax.ShapeDtypeStruct((N, nherbs), jnp.float32),
        compiler_params=pltpu.CompilerParams(
            dimension_semantics=("parallel",)),
    )(s2, b2.reshape(1, dim), Wsh.T, bsh.reshape(1, nherbs),
      Whc.T, bhc.reshape(1, nherbs), adj)

    return (out[:NUM_SYMPS], out[NUM_SYMPS:])


# speedup vs baseline: 1.0028x; 1.0028x over previous
import functools

import jax
import jax.numpy as jnp
from jax.experimental import pallas as pl
from jax.experimental.pallas import tpu as pltpu

N = 10000
NUM_SYMPS = 360
ROW_BLOCK = 400

def _dot(a, b):
    return jnp.dot(a, b, precision=jax.lax.Precision.DEFAULT,
                   preferred_element_type=jnp.float32)


def _s1_kernel(x_ref, w1_ref, s1_ref):
    s1_ref[...] = _dot(x_ref[...], w1_ref[...])


def _pass1_kernel(s1_ref, b1_ref, w2_ref, adj_ref, s2_ref):
    h = jnp.maximum(_dot(adj_ref[...], s1_ref[...]) + b1_ref[...], 0.0)
    s2_ref[...] = _dot(h, w2_ref[...])


def _pass2_kernel(s2_ref, b2_ref, wsh_t_ref, bsh_ref, whc_t_ref, bhc_ref,
                  adj_ref, out_ref):
    h2 = _dot(adj_ref[...], s2_ref[...]) + b2_ref[...]
    logits_s = _dot(h2, wsh_t_ref[...]) + bsh_ref[...]
    logits_h = _dot(h2, whc_t_ref[...]) + bhc_ref[...]
    rows = (pl.program_id(0) * ROW_BLOCK
            + jax.lax.broadcasted_iota(jnp.int32, (ROW_BLOCK, 1), 0))
    out_ref[...] = jax.nn.sigmoid(
        jnp.where(rows < NUM_SYMPS, logits_s, logits_h))


@jax.jit
def kernel(x, adj, W1, b1, W2, b2, Wsh, bsh, Whc, bhc):
    nfeat = x.shape[1]
    nhid = W1.shape[1]
    dim = W2.shape[1]
    nherbs = Wsh.shape[0]
    num_blocks = N // ROW_BLOCK

    s1 = pl.pallas_call(
        _s1_kernel,
        out_shape=jax.ShapeDtypeStruct((N, nhid), jnp.float32),
    )(x, W1)

    full = lambda shape: pl.BlockSpec(shape, lambda i: (0, 0))

    s2 = pl.pallas_call(
        _pass1_kernel,
        grid=(num_blocks,),
        in_specs=[
            full((N, nhid)),
            full((1, nhid)),
            full((nhid, dim)),
            pl.BlockSpec((ROW_BLOCK, N), lambda i: (i, 0)),
        ],
        out_specs=pl.BlockSpec((ROW_BLOCK, dim), lambda i: (i, 0)),
        out_shape=jax.ShapeDtypeStruct((N, dim), jnp.float32),
        compiler_params=pltpu.CompilerParams(
            dimension_semantics=("parallel",)),
    )(s1, b1.reshape(1, nhid), W2, adj)

    out = pl.pallas_call(
        _pass2_kernel,
        grid=(num_blocks,),
        in_specs=[
            full((N, dim)),
            full((1, dim)),
            full((dim, nherbs)),
            full((1, nherbs)),
            full((dim, nherbs)),
            full((1, nherbs)),
            pl.BlockSpec((ROW_BLOCK, N), lambda i: (i, 0)),
        ],
        out_specs=pl.BlockSpec((ROW_BLOCK, nherbs), lambda i: (i, 0)),
        out_shape=jax.ShapeDtypeStruct((N, nherbs), jnp.float32),
        compiler_params=pltpu.CompilerParams(
            dimension_semantics=("parallel",)),
    )(s2, b2.reshape(1, dim), Wsh.T, bsh.reshape(1, nherbs),
      Whc.T, bhc.reshape(1, nherbs), adj)

    return (out[:NUM_SYMPS], out[NUM_SYMPS:])
